# baseline (speedup 1.0000x reference)
import functools

import jax
import jax.numpy as jnp
from jax import lax
from jax.experimental import pallas as pl
from jax.experimental.pallas import tpu as pltpu
from jax.experimental.pallas import tpu_sc as plsc

NC = 2
NS = 16
L = 16
NW = NC * NS

E = 320000
N = 10000
D = 128
C = 128
PER_W = E // NW
NFULL = PER_W // C
TAIL = PER_W - NFULL * C
CW = C // 2
PAD_W = (NFULL + 1) * CW

_GATHER_DNUMS = lax.GatherDimensionNumbers(
    offset_dims=(), collapsed_slice_dims=(0,), start_index_map=(0,))


def _lane_perm(v, idx16):
    return lax.gather(v, idx16[:, None], _GATHER_DNUMS, (1,),
                      mode=lax.GatherScatterMode.PROMISE_IN_BOUNDS)


def _treesum(vs):
    while len(vs) > 1:
        vs = [a + b for a, b in zip(vs[0::2], vs[1::2])]
    return vs[0]


def _edge_partials(bs, bc, j):
    ps = [bs[j, pl.ds(q * 32, 32)] * bc[j, pl.ds(q * 32, 32)]
          for q in range(D // 32)]
    s = _treesum(ps)
    a, b = plsc.unpack(s, format=plsc.PackFormat.INTERLEAVED)
    return a + b


def _combine(x, y, lvl, evens, perms):
    pk = perms[lvl]
    sx = x + _lane_perm(x, pk)
    sy = y + _lane_perm(y, pk)
    return jnp.where(evens[lvl], sx, sy)


def _group16(bs, bc, g, evens, perms):
    stack = []
    for jj in range(16):
        v = _edge_partials(bs, bc, g * 16 + jj)
        lvl = 0
        while stack and stack[-1][0] == lvl:
            _, u = stack.pop()
            v = _combine(u, v, lvl, evens, perms)
            lvl += 1
        stack.append((lvl, v))
    return stack[0][1]


def _pack_idx(idx):
    per = idx.reshape(NW, PER_W)
    pad = jnp.zeros((NW, (NFULL + 1) * C - PER_W), jnp.int32)
    x = jnp.concatenate([per, pad], axis=1).reshape(NW, NFULL + 1, C // 32, 2, 16)
    w = x[:, :, :, 0, :] | (x[:, :, :, 1, :] << 16)
    return w.reshape(NW * PAD_W)


def _make_decode():
    mesh = plsc.VectorSubcoreMesh(core_axis_name="c", subcore_axis_name="s")

    @functools.partial(
        pl.kernel,
        mesh=mesh,
        compiler_params=pltpu.CompilerParams(
            needs_layout_passes=False, use_tc_tiling_on_sc=False),
        out_type=jax.ShapeDtypeStruct((E,), jnp.float32),
        scratch_types=[
            pltpu.VMEM((PAD_W,), jnp.int32),
            pltpu.VMEM((PAD_W,), jnp.int32),
            pltpu.VMEM((C,), jnp.int32),
            pltpu.VMEM((C,), jnp.int32),
            pltpu.VMEM((C,), jnp.int32),
            pltpu.VMEM((C,), jnp.int32),
            pltpu.VMEM((C, D), jnp.bfloat16),
            pltpu.VMEM((C, D), jnp.bfloat16),
            pltpu.VMEM((C, D), jnp.bfloat16),
            pltpu.VMEM((C, D), jnp.bfloat16),
            pltpu.VMEM((C,), jnp.float32),
            pltpu.VMEM((C,), jnp.float32),
            pltpu.VMEM_SHARED((N, D), jnp.bfloat16),
            pltpu.VMEM_SHARED((N, D), jnp.bfloat16),
            pltpu.SemaphoreType.DMA,
            pltpu.SemaphoreType.DMA,
            pltpu.SemaphoreType.DMA,
            pltpu.SemaphoreType.DMA,
            pltpu.SemaphoreType.DMA,
            pltpu.SemaphoreType.DMA,
        ],
    )
    def decode(zs_hbm, zc_hbm, sip_hbm, dip_hbm, out_hbm,
               idxp_s, idxp_d, iu_s0, iu_d0, iu_s1, iu_d1,
               bs0, bc0, bs1, bc1, o0, o1, zs_sh, zc_sh,
               sem_s0, sem_c0, sem_s1, sem_c1, sem_o0, sem_o1):
        wid = lax.axis_index("s") * NC + lax.axis_index("c")
        base = wid * PER_W
        pltpu.sync_copy(sip_hbm.at[pl.ds(wid * PAD_W, PAD_W)], idxp_s)
        pltpu.sync_copy(dip_hbm.at[pl.ds(wid * PAD_W, PAD_W)], idxp_d)

        sid = lax.axis_index("s")
        rows = N // NS
        r0 = sid * rows
        pltpu.sync_copy(zs_hbm.at[pl.ds(r0, rows)], zs_sh.at[pl.ds(r0, rows)])
        pltpu.sync_copy(zc_hbm.at[pl.ds(r0, rows)], zc_sh.at[pl.ds(r0, rows)])
        plsc.subcore_barrier()

        lane = lax.iota(jnp.int32, 16)
        perms = [lane ^ k for k in (1, 2, 4, 8)]
        evens = [(lane & k) == 0 for k in (1, 2, 4, 8)]

        def unpack_idx(idxp, iu, i, groups):
            for j in range(groups):
                w = idxp[pl.ds(i * CW + j * 16, 16)]
                iu[pl.ds(j * 32, 16)] = w & jnp.int32(0xFFFF)
                iu[pl.ds(j * 32 + 16, 16)] = w >> 16

        def start(i, iu_s, iu_d, bs, bc, sem_s, sem_c):
            unpack_idx(idxp_s, iu_s, i, C // 32)
            unpack_idx(idxp_d, iu_d, i, C // 32)
            pltpu.async_copy(zs_sh.at[iu_s], bs, sem_s)
            pltpu.async_copy(zc_sh.at[iu_d], bc, sem_c)

        def wait(bs, bc, sem_s, sem_c):
            pltpu.make_async_copy(zs_hbm.at[pl.ds(0, C)], bs, sem_s).wait()
            pltpu.make_async_copy(zc_hbm.at[pl.ds(0, C)], bc, sem_c).wait()

        def compute(i, bs, bc, o, sem_o):
            def gbody(g, carry):
                vec = _group16(bs, bc, g, evens, perms)
                sig = 1.0 / (1.0 + jnp.exp(-vec))
                o[pl.ds(g * 16, 16)] = sig
                return carry
            lax.fori_loop(0, C // 16, gbody, 0)
            pltpu.async_copy(o, out_hbm.at[pl.ds(base + i * C, C)], sem_o)

        def wait_out(o, sem_o):
            pltpu.make_async_copy(o, out_hbm.at[pl.ds(0, C)], sem_o).wait()

        start(0, iu_s0, iu_d0, bs0, bc0, sem_s0, sem_c0)
        start(1, iu_s1, iu_d1, bs1, bc1, sem_s1, sem_c1)
        wait(bs0, bc0, sem_s0, sem_c0)
        compute(0, bs0, bc0, o0, sem_o0)
        start(2, iu_s0, iu_d0, bs0, bc0, sem_s0, sem_c0)
        wait(bs1, bc1, sem_s1, sem_c1)
        compute(1, bs1, bc1, o1, sem_o1)

        def body(g, carry):
            i = 2 * g + 2
            start(i + 1, iu_s1, iu_d1, bs1, bc1, sem_s1, sem_c1)
            wait(bs0, bc0, sem_s0, sem_c0)
            wait_out(o0, sem_o0)
            compute(i, bs0, bc0, o0, sem_o0)
            start(i + 2, iu_s0, iu_d0, bs0, bc0, sem_s0, sem_c0)
            wait(bs1, bc1, sem_s1, sem_c1)
            wait_out(o1, sem_o1)
            compute(i + 1, bs1, bc1, o1, sem_o1)
            return carry

        lax.fori_loop(0, (NFULL - 2) // 2, body, 0)

        wait(bs0, bc0, sem_s0, sem_c0)
        wait_out(o0, sem_o0)
        vec = _group16(bs0, bc0, 0, evens, perms)
        o0[pl.ds(0, TAIL)] = 1.0 / (1.0 + jnp.exp(-vec))
        pltpu.async_copy(o0.at[pl.ds(0, TAIL)],
                         out_hbm.at[pl.ds(base + NFULL * C, TAIL)], sem_o0)
        pltpu.make_async_copy(o0.at[pl.ds(0, TAIL)],
                              out_hbm.at[pl.ds(0, TAIL)], sem_o0).wait()
        wait_out(o1, sem_o1)

    return decode


_decode = _make_decode()


def kernel(z_source, z_child, edge_index):
    src_idx = edge_index[0].astype(jnp.int32)
    dst_idx = edge_index[1].astype(jnp.int32)
    zs = z_source.astype(jnp.bfloat16)
    zc = z_child.astype(jnp.bfloat16)
    return _decode(zs, zc, _pack_idx(src_idx), _pack_idx(dst_idx))


# speedup vs baseline: 1.3276x; 1.3276x over previous
import functools

import jax
import jax.numpy as jnp
from jax import lax
from jax.experimental import pallas as pl
from jax.experimental.pallas import tpu as pltpu
from jax.experimental.pallas import tpu_sc as plsc

NC = 2
NS = 16
L = 16
NW = NC * NS

E = 320000
N = 10000
D = 128
C = 80
PER_W = E // NW
NCHUNK = PER_W // C

_GATHER_DNUMS = lax.GatherDimensionNumbers(
    offset_dims=(), collapsed_slice_dims=(0,), start_index_map=(0,))


def _lane_perm(v, idx16):
    return lax.gather(v, idx16[:, None], _GATHER_DNUMS, (1,),
                      mode=lax.GatherScatterMode.PROMISE_IN_BOUNDS)


def _treesum(vs):
    while len(vs) > 1:
        vs = [a + b for a, b in zip(vs[0::2], vs[1::2])]
    return vs[0]


def _edge_partials(bs, bc, j):
    ps = [bs[j, pl.ds(q * 32, 32)] * bc[j, pl.ds(q * 32, 32)]
          for q in range(D // 32)]
    s = _treesum(ps)
    a, b = plsc.unpack(s, format=plsc.PackFormat.INTERLEAVED)
    return a + b


def _combine(x, y, lvl, evens, perms):
    pk = perms[lvl]
    sx = x + _lane_perm(x, pk)
    sy = y + _lane_perm(y, pk)
    return jnp.where(evens[lvl], sx, sy)


def _group16(bs, bc, g, evens, perms):
    stack = []
    for jj in range(16):
        v = _edge_partials(bs, bc, g * 16 + jj)
        lvl = 0
        while stack and stack[-1][0] == lvl:
            _, u = stack.pop()
            v = _combine(u, v, lvl, evens, perms)
            lvl += 1
        stack.append((lvl, v))
    return stack[0][1]


def _make_decode():
    mesh = plsc.VectorSubcoreMesh(core_axis_name="c", subcore_axis_name="s")

    @functools.partial(
        pl.kernel,
        mesh=mesh,
        compiler_params=pltpu.CompilerParams(
            needs_layout_passes=False, use_tc_tiling_on_sc=False),
        out_type=jax.ShapeDtypeStruct((E,), jnp.float32),
        scratch_types=[
            pltpu.VMEM((PER_W,), jnp.int32),
            pltpu.VMEM((PER_W,), jnp.int32),
            pltpu.VMEM((C, D), jnp.bfloat16),
            pltpu.VMEM((C, D), jnp.bfloat16),
            pltpu.VMEM((C, D), jnp.bfloat16),
            pltpu.VMEM((C, D), jnp.bfloat16),
            pltpu.VMEM((C,), jnp.float32),
            pltpu.VMEM((C,), jnp.float32),
            pltpu.VMEM_SHARED((N, D), jnp.bfloat16),
            pltpu.VMEM_SHARED((N, D), jnp.bfloat16),
            pltpu.SemaphoreType.DMA,
            pltpu.SemaphoreType.DMA,
            pltpu.SemaphoreType.DMA,
            pltpu.SemaphoreType.DMA,
            pltpu.SemaphoreType.DMA,
            pltpu.SemaphoreType.DMA,
        ],
    )
    def decode(zs_hbm, zc_hbm, si_hbm, di_hbm, out_hbm,
               idx_s, idx_d, bs0, bc0, bs1, bc1, o0, o1, zs_sh, zc_sh,
               sem_s0, sem_c0, sem_s1, sem_c1, sem_o0, sem_o1):
        wid = lax.axis_index("s") * NC + lax.axis_index("c")
        base = wid * PER_W
        pltpu.sync_copy(si_hbm.at[pl.ds(base, PER_W)], idx_s)
        pltpu.sync_copy(di_hbm.at[pl.ds(base, PER_W)], idx_d)

        sid = lax.axis_index("s")
        rows = N // NS
        r0 = sid * rows
        pltpu.sync_copy(zs_hbm.at[pl.ds(r0, rows)], zs_sh.at[pl.ds(r0, rows)])
        pltpu.sync_copy(zc_hbm.at[pl.ds(r0, rows)], zc_sh.at[pl.ds(r0, rows)])
        plsc.subcore_barrier()

        lane = lax.iota(jnp.int32, 16)
        perms = [lane ^ k for k in (1, 2, 4, 8)]
        evens = [(lane & k) == 0 for k in (1, 2, 4, 8)]

        def start(i, bs, bc, sem_s, sem_c):
            pltpu.async_copy(zs_sh.at[idx_s.at[pl.ds(i * C, C)]], bs, sem_s)
            pltpu.async_copy(zc_sh.at[idx_d.at[pl.ds(i * C, C)]], bc, sem_c)

        def wait(bs, bc, sem_s, sem_c):
            pltpu.make_async_copy(zs_hbm.at[pl.ds(0, C)], bs, sem_s).wait()
            pltpu.make_async_copy(zc_hbm.at[pl.ds(0, C)], bc, sem_c).wait()

        def compute(i, bs, bc, o, sem_o):
            def gbody(g, carry):
                vec = _group16(bs, bc, g, evens, perms)
                sig = 1.0 / (1.0 + jnp.exp(-vec))
                o[pl.ds(g * 16, 16)] = sig
                return carry
            lax.fori_loop(0, C // 16, gbody, 0)
            pltpu.async_copy(o, out_hbm.at[pl.ds(base + i * C, C)], sem_o)

        def wait_out(o, sem_o):
            pltpu.make_async_copy(o, out_hbm.at[pl.ds(0, C)], sem_o).wait()

        start(0, bs0, bc0, sem_s0, sem_c0)
        start(1, bs1, bc1, sem_s1, sem_c1)
        wait(bs0, bc0, sem_s0, sem_c0)
        compute(0, bs0, bc0, o0, sem_o0)
        start(2, bs0, bc0, sem_s0, sem_c0)
        wait(bs1, bc1, sem_s1, sem_c1)
        compute(1, bs1, bc1, o1, sem_o1)

        def body(g, carry):
            i = 2 * g + 2
            start(i + 1, bs1, bc1, sem_s1, sem_c1)
            wait(bs0, bc0, sem_s0, sem_c0)
            wait_out(o0, sem_o0)
            compute(i, bs0, bc0, o0, sem_o0)
            start(i + 2, bs0, bc0, sem_s0, sem_c0)
            wait(bs1, bc1, sem_s1, sem_c1)
            wait_out(o1, sem_o1)
            compute(i + 1, bs1, bc1, o1, sem_o1)
            return carry

        lax.fori_loop(0, (NCHUNK - 3) // 2, body, 0)
        wait(bs0, bc0, sem_s0, sem_c0)
        wait_out(o0, sem_o0)
        compute(NCHUNK - 1, bs0, bc0, o0, sem_o0)
        wait_out(o0, sem_o0)
        wait_out(o1, sem_o1)

    return decode


_decode = _make_decode()


def kernel(z_source, z_child, edge_index):
    src_idx = edge_index[0].astype(jnp.int32)
    dst_idx = edge_index[1].astype(jnp.int32)
    zs = z_source.astype(jnp.bfloat16)
    zc = z_child.astype(jnp.bfloat16)
    return _decode(zs, zc, src_idx, dst_idx)
